# baseline (speedup 1.0000x reference)
import functools

import jax
import jax.numpy as jnp
from jax import lax
from jax.experimental import pallas as pl
from jax.experimental.pallas import tpu as pltpu
from jax.experimental.pallas import tpu_sc as plsc

NPTS = 50000
NBATCH = 16
KNN = 16
SD = 4
PD = 8
PDP = 16
MSGW = 128
EMBD = 32
CH = 4096
NWORK = 32
QPW = 1568
ROWS = 2000


def _elu(h):
    small = jnp.abs(h) < 1e-5
    em1 = jnp.where(small, h + 0.5 * h * h, jnp.exp(h) - 1.0)
    return jnp.where(h > 0.0, h, em1)


def _mm(a, b):
    return jnp.dot(a, b)


def _full_spec(shape):
    return pl.BlockSpec(shape, lambda i: tuple(0 for _ in shape))


def _row_spec(cols):
    return pl.BlockSpec((ROWS, cols), lambda i: (i, 0))


def _tc_lin(h, w, b):
    def body(h_ref, w_ref, b_ref, o_ref):
        o_ref[...] = _mm(h_ref[...], w_ref[...]) + b_ref[...]

    return pl.pallas_call(
        body,
        grid=(NPTS // ROWS,),
        in_specs=[_row_spec(h.shape[1]), _full_spec(w.shape),
                  _full_spec(b.shape)],
        out_specs=_row_spec(w.shape[1]),
        out_shape=jax.ShapeDtypeStruct((NPTS, w.shape[1]), jnp.float32),
    )(h, w, b)


def _tc_emb3(h, p):
    def body(h_ref, w4, b4, wp, bp, ws, bs, msg_ref, s_ref):
        e = _mm(h_ref[...], w4[...]) + b4[...]
        msg_ref[...] = _mm(e, wp[...]) + bp[...]
        s_ref[...] = _mm(e, ws[...]) + bs[...]

    args = [h] + p
    return pl.pallas_call(
        body,
        grid=(NPTS // ROWS,),
        in_specs=[_row_spec(h.shape[1])] + [_full_spec(a.shape) for a in p],
        out_specs=[_row_spec(MSGW), _row_spec(SD)],
        out_shape=[jax.ShapeDtypeStruct((NPTS, MSGW), jnp.float32),
                   jax.ShapeDtypeStruct((NPTS, SD), jnp.float32)],
    )(*args)


def _tc_mid(agg16, p):
    def body(a_ref, wo, bo, wp, bp, ws, bs, msg_ref, s_ref):
        e = _mm(a_ref[...], wo[...]) + bo[...]
        msg_ref[...] = _mm(e, wp[...]) + bp[...]
        s_ref[...] = _mm(e, ws[...]) + bs[...]

    args = [agg16] + p
    return pl.pallas_call(
        body,
        grid=(NPTS // ROWS,),
        in_specs=[_row_spec(PDP)] + [_full_spec(a.shape) for a in p],
        out_specs=[_row_spec(MSGW), _row_spec(SD)],
        out_shape=[jax.ShapeDtypeStruct((NPTS, MSGW), jnp.float32),
                   jax.ShapeDtypeStruct((NPTS, SD), jnp.float32)],
    )(*args)


def _tc_post(agg16, x, p):
    def body(a_ref, x_ref, wo, bo,
             w1x, w1e, b1, w2, b2, w3, b3, w4, b4,
             v1x, v1p, c1, v2, c2, v3, c3, v4, c4,
             id_ref, p4_ref):
        xb = x_ref[...]
        e = _mm(a_ref[...], wo[...]) + bo[...]
        h = _elu(_mm(xb, w1x[...]) + _mm(e, w1e[...]) + b1[...])
        h = _elu(_mm(h, w2[...]) + b2[...])
        h = _elu(_mm(h, w3[...]) + b3[...])
        pid = _mm(h, w4[...]) + b4[...]
        id_ref[...] = pid
        h = _elu(_mm(xb, v1x[...]) + _mm(pid, v1p[...]) + c1[...])
        h = _elu(_mm(h, v2[...]) + c2[...])
        h = _elu(_mm(h, v3[...]) + c3[...])
        p4_ref[...] = _mm(h, v4[...]) + c4[...]

    args = [agg16, x] + p
    return pl.pallas_call(
        body,
        grid=(NPTS // ROWS,),
        in_specs=[_row_spec(PDP), _row_spec(x.shape[1])]
        + [_full_spec(a.shape) for a in p],
        out_specs=[_row_spec(6), _row_spec(6)],
        out_shape=[jax.ShapeDtypeStruct((NPTS, 6), jnp.float32),
                   jax.ShapeDtypeStruct((NPTS, 6), jnp.float32)],
    )(*args)


def _sc_compiler_params():
    import dataclasses
    cp = pltpu.CompilerParams()
    if "needs_layout_passes" in pltpu.CompilerParams.__dataclass_fields__:
        cp = dataclasses.replace(cp, needs_layout_passes=False)
    return cp


def _sc_gravnet(s_flat, msg16, batch, seg):
    mesh = plsc.VectorSubcoreMesh(core_axis_name="c", subcore_axis_name="s")
    INF = jnp.float32(1e30)

    @functools.partial(
        pl.kernel,
        out_type=jax.ShapeDtypeStruct((NPTS * PDP,), jnp.float32),
        mesh=mesh,
        compiler_params=_sc_compiler_params(),
        scratch_types=[
            pltpu.VMEM((CH * SD,), jnp.float32),
            pltpu.VMEM((16 * SD,), jnp.float32),
            pltpu.VMEM((16,), jnp.int32),
            pltpu.VMEM((32,), jnp.int32),
            pltpu.VMEM((128,), jnp.int32),
            pltpu.VMEM((128,), jnp.int32),
            pltpu.VMEM((128, MSGW), jnp.float32),
            pltpu.VMEM((128, MSGW), jnp.float32),
            pltpu.VMEM((256,), jnp.float32),
            pltpu.VMEM((16, 16), jnp.float32),
            pltpu.VMEM((16, 16), jnp.int32),
        ],
    )
    def knl(s_hbm, msg_hbm, batch_hbm, seg_hbm, agg_hbm,
            cand, qsb, qbb, segb, idxa, idxb, rowsa, rowsb, outb,
            bestd, besti):
        cid = lax.axis_index("c")
        sid = lax.axis_index("s")
        wid = sid * 2 + cid
        pltpu.sync_copy(seg_hbm, segb)
        q_start = wid * QPW
        ngroups = jnp.minimum(QPW, NPTS - q_start) // 16
        lane = lax.broadcasted_iota(jnp.int32, (16,), 0)

        def group_body(g, carry):
            q0 = q_start + g * 16
            pltpu.sync_copy(s_hbm.at[pl.ds(q0 * SD, 16 * SD)], qsb)
            pltpu.sync_copy(batch_hbm.at[pl.ds(q0, 16)], qbb)
            qb = qbb[...]
            lo = plsc.load_gather(segb, [qb])
            hi = plsc.load_gather(segb, [qb + 1])
            qs = [plsc.load_gather(qsb, [lane * SD + d]) for d in range(SD)]
            u_lo = jnp.min(lo)
            u_hi = jnp.max(hi)
            mx_lo = jnp.max(lo)
            mn_hi = jnp.min(hi)
            c_base = (u_lo // 8) * 8
            nch = (u_hi - c_base + (CH - 1)) // CH

            for k in range(16):
                bestd[k, :] = jnp.full((16,), INF, jnp.float32)
                besti[k, :] = jnp.zeros((16,), jnp.int32)

            def insert4(dists, c_first):
                obd = [bestd[k, :] for k in range(16)]
                obi = [besti[k, :] for k in range(16)]
                for t in range(4):
                    dist = dists[t]
                    cvec = jnp.full((16,), c_first + t, jnp.int32)
                    m = [dist < obd[k] for k in range(16)]
                    nd = [None] * 16
                    ni = [None] * 16
                    for k in range(15, -1, -1):
                        if k == 0:
                            sd_, si_ = dist, cvec
                        else:
                            sd_ = jnp.where(m[k - 1], obd[k - 1], dist)
                            si_ = jnp.where(m[k - 1], obi[k - 1], cvec)
                        nd[k] = jnp.where(m[k], sd_, obd[k])
                        ni[k] = jnp.where(m[k], si_, obi[k])
                    obd, obi = nd, ni
                for k in range(16):
                    bestd[k, :] = obd[k]
                    besti[k, :] = obi[k]

            def make_body(c0, checked):
                def body(jj, carry):
                    cv = cand[pl.ds(jj * 16, 16)]
                    worst = bestd[15, :]
                    dists = []
                    anym = None
                    for t in range(4):
                        acc = None
                        for d in range(SD):
                            b = jnp.full((16,), cv[4 * t + d])
                            tt = qs[d] - b
                            acc = tt * tt if acc is None else acc + tt * tt
                        if checked:
                            cvec = jnp.full((16,), c0 + jj * 4 + t, jnp.int32)
                            valid = (cvec >= lo) & (cvec < hi)
                            acc = jnp.where(valid, acc, INF)
                        m = acc < worst
                        anym = m if anym is None else (anym | m)
                        dists.append(acc)
                    cnt = plsc.all_reduce_population_count(anym)

                    @pl.when(cnt[0] >= 0)
                    def _():
                        insert4(dists, c0 + jj * 4)

                    return carry
                return body

            def chunk_body(ci, carry):
                c0 = c_base + ci * CH
                pltpu.sync_copy(s_hbm.at[pl.ds(c0 * SD, CH * SD)], cand)
                rel_lo = jnp.clip(u_lo - c0, 0, CH)
                rel_ml = jnp.clip(mx_lo - c0, 0, CH)
                rel_mh = jnp.clip(mn_hi - c0, 0, CH)
                rel_hi = jnp.clip(u_hi - c0, 0, CH)
                b0 = rel_lo // 4
                b3 = (rel_hi + 3) // 4
                b1 = jnp.minimum((rel_ml + 3) // 4, b3)
                b1 = jnp.maximum(b0, b1)
                b2 = jnp.maximum(jnp.minimum(rel_mh // 4, b3), b1)
                lax.fori_loop(b0, b1, make_body(c0, True), 0)
                lax.fori_loop(b1, b2, make_body(c0, False), 0)
                lax.fori_loop(b2, b3, make_body(c0, True), 0)
                return carry

            lax.fori_loop(0, nch, chunk_body, 0)
            bd = [bestd[k, :] for k in range(16)]
            bi = [besti[k, :] for k in range(16)]
            ews = [jnp.exp(-10.0 * bd[k]) for k in range(16)]
            for k in range(16):
                if k < 8:
                    idxa[pl.ds(k * 16, 16)] = bi[k]
                else:
                    idxb[pl.ds((k - 8) * 16, 16)] = bi[k]
            pltpu.sync_copy(msg_hbm.at[idxa], rowsa)
            pltpu.sync_copy(msg_hbm.at[idxb], rowsb)
            for ql in range(16):
                a = jnp.zeros((16,), jnp.float32)
                for k in range(16):
                    rref = rowsa if k < 8 else rowsb
                    w = jnp.full((16,), ews[k][ql])
                    a = a + w * rref[(k % 8) * 16 + ql, pl.ds(0, 16)]
                outb[pl.ds(ql * 16, 16)] = a
            pltpu.sync_copy(outb, agg_hbm.at[pl.ds(q0 * PDP, 256)])
            return carry

        lax.fori_loop(0, ngroups, group_body, 0)

    return knl(s_flat, msg16, batch, seg)


def _prep_lin(p):
    return jnp.asarray(p["W"]).T, jnp.asarray(p["b"])[None, :]


def _prep_lin_padout(p, out_pad):
    w, b = _prep_lin(p)
    w = jnp.pad(w, ((0, 0), (0, out_pad - w.shape[1])))
    b = jnp.pad(b, ((0, 0), (0, out_pad - b.shape[1])))
    return w, b


def kernel(x, batch, params):
    batch = batch.astype(jnp.int32)
    seg = jnp.searchsorted(
        batch, jnp.arange(NBATCH + 1, dtype=jnp.int32), side="left"
    ).astype(jnp.int32)
    seg = jnp.pad(seg, (0, 32 - (NBATCH + 1)))

    nn1 = params["nn1"]
    convs = params["convs"]
    nn2 = params["nn2"]
    nn3 = params["nn3"]

    h = x
    for lp in nn1[:-1]:
        w, b = _prep_lin(lp)
        h = jax.nn.elu(_tc_lin(h, w, b))
    w4, b4 = _prep_lin(nn1[-1])
    wp, bp = _prep_lin_padout(convs[0]["lin_p"], MSGW)
    ws, bs = _prep_lin(convs[0]["lin_s"])
    msg16, s = _tc_emb3(h, [w4, b4, wp, bp, ws, bs])

    for i in range(3):
        s_flat = jnp.pad(s, ((0, CH), (0, 0))).reshape(-1)
        agg_flat = _sc_gravnet(s_flat, msg16, batch, seg)
        agg16 = agg_flat.reshape(NPTS, PDP)
        wo, bo = _prep_lin(convs[i]["lin_out"])
        wo = jnp.pad(wo, ((0, PDP - PD), (0, 0)))
        if i < 2:
            wp, bp = _prep_lin_padout(convs[i + 1]["lin_p"], MSGW)
            ws, bs = _prep_lin(convs[i + 1]["lin_s"])
            msg16, s = _tc_mid(agg16, [wo, bo, wp, bp, ws, bs])
        else:
            w1, b1 = _prep_lin(nn2[0])
            p_post = [wo, bo,
                      w1[:12], w1[12:], b1]
            for lp in nn2[1:]:
                w, b = _prep_lin(lp)
                p_post += [w, b]
            v1, c1 = _prep_lin(nn3[0])
            p_post += [v1[:12], v1[12:], c1]
            for lp in nn3[1:]:
                w, b = _prep_lin(lp)
                p_post += [w, b]
            preds_id, preds_p4 = _tc_post(agg16, x, p_post)
    return preds_id, preds_p4


# speedup vs baseline: 1.3924x; 1.3924x over previous
import functools

import jax
import jax.numpy as jnp
from jax import lax
from jax.experimental import pallas as pl
from jax.experimental.pallas import tpu as pltpu
from jax.experimental.pallas import tpu_sc as plsc

NPTS = 50000
NBATCH = 16
KNN = 16
SD = 4
PD = 8
PDP = 16
MSGW = 128
EMBD = 32
CH = 4096
NWORK = 32
QPW = 1568
ROWS = 2000


def _elu(h):
    small = jnp.abs(h) < 1e-5
    em1 = jnp.where(small, h + 0.5 * h * h, jnp.exp(h) - 1.0)
    return jnp.where(h > 0.0, h, em1)


def _mm(a, b):
    return jnp.dot(a, b)


def _full_spec(shape):
    return pl.BlockSpec(shape, lambda i: tuple(0 for _ in shape))


def _row_spec(cols):
    return pl.BlockSpec((ROWS, cols), lambda i: (i, 0))


def _tc_lin(h, w, b):
    def body(h_ref, w_ref, b_ref, o_ref):
        o_ref[...] = _mm(h_ref[...], w_ref[...]) + b_ref[...]

    return pl.pallas_call(
        body,
        grid=(NPTS // ROWS,),
        in_specs=[_row_spec(h.shape[1]), _full_spec(w.shape),
                  _full_spec(b.shape)],
        out_specs=_row_spec(w.shape[1]),
        out_shape=jax.ShapeDtypeStruct((NPTS, w.shape[1]), jnp.float32),
    )(h, w, b)


def _tc_emb3(h, p):
    def body(h_ref, w4, b4, wp, bp, ws, bs, msg_ref, s_ref):
        e = _mm(h_ref[...], w4[...]) + b4[...]
        msg_ref[...] = _mm(e, wp[...]) + bp[...]
        s_ref[...] = _mm(e, ws[...]) + bs[...]

    args = [h] + p
    return pl.pallas_call(
        body,
        grid=(NPTS // ROWS,),
        in_specs=[_row_spec(h.shape[1])] + [_full_spec(a.shape) for a in p],
        out_specs=[_row_spec(MSGW), _row_spec(SD)],
        out_shape=[jax.ShapeDtypeStruct((NPTS, MSGW), jnp.float32),
                   jax.ShapeDtypeStruct((NPTS, SD), jnp.float32)],
    )(*args)


def _tc_mid(agg16, p):
    def body(a_ref, wo, bo, wp, bp, ws, bs, msg_ref, s_ref):
        e = _mm(a_ref[...], wo[...]) + bo[...]
        msg_ref[...] = _mm(e, wp[...]) + bp[...]
        s_ref[...] = _mm(e, ws[...]) + bs[...]

    args = [agg16] + p
    return pl.pallas_call(
        body,
        grid=(NPTS // ROWS,),
        in_specs=[_row_spec(PDP)] + [_full_spec(a.shape) for a in p],
        out_specs=[_row_spec(MSGW), _row_spec(SD)],
        out_shape=[jax.ShapeDtypeStruct((NPTS, MSGW), jnp.float32),
                   jax.ShapeDtypeStruct((NPTS, SD), jnp.float32)],
    )(*args)


def _tc_post(agg16, x, p):
    def body(a_ref, x_ref, wo, bo,
             w1x, w1e, b1, w2, b2, w3, b3, w4, b4,
             v1x, v1p, c1, v2, c2, v3, c3, v4, c4,
             id_ref, p4_ref):
        xb = x_ref[...]
        e = _mm(a_ref[...], wo[...]) + bo[...]
        h = _elu(_mm(xb, w1x[...]) + _mm(e, w1e[...]) + b1[...])
        h = _elu(_mm(h, w2[...]) + b2[...])
        h = _elu(_mm(h, w3[...]) + b3[...])
        pid = _mm(h, w4[...]) + b4[...]
        id_ref[...] = pid
        h = _elu(_mm(xb, v1x[...]) + _mm(pid, v1p[...]) + c1[...])
        h = _elu(_mm(h, v2[...]) + c2[...])
        h = _elu(_mm(h, v3[...]) + c3[...])
        p4_ref[...] = _mm(h, v4[...]) + c4[...]

    args = [agg16, x] + p
    return pl.pallas_call(
        body,
        grid=(NPTS // ROWS,),
        in_specs=[_row_spec(PDP), _row_spec(x.shape[1])]
        + [_full_spec(a.shape) for a in p],
        out_specs=[_row_spec(6), _row_spec(6)],
        out_shape=[jax.ShapeDtypeStruct((NPTS, 6), jnp.float32),
                   jax.ShapeDtypeStruct((NPTS, 6), jnp.float32)],
    )(*args)


def _lane_bcast(v, idx):
    dn = lax.GatherDimensionNumbers(
        offset_dims=(), collapsed_slice_dims=(0,), start_index_map=(0,))
    return lax.gather(v, idx[:, None], dn, slice_sizes=(1,),
                      mode=lax.GatherScatterMode.PROMISE_IN_BOUNDS)


def _sc_compiler_params():
    import dataclasses
    cp = pltpu.CompilerParams()
    if "needs_layout_passes" in pltpu.CompilerParams.__dataclass_fields__:
        cp = dataclasses.replace(cp, needs_layout_passes=False)
    return cp


def _sc_gravnet(s_flat, msg16, batch, seg):
    mesh = plsc.VectorSubcoreMesh(core_axis_name="c", subcore_axis_name="s")
    INF = jnp.float32(1e30)

    @functools.partial(
        pl.kernel,
        out_type=jax.ShapeDtypeStruct((NPTS * PDP,), jnp.float32),
        mesh=mesh,
        compiler_params=_sc_compiler_params(),
        scratch_types=[
            pltpu.VMEM((CH * SD,), jnp.float32),
            pltpu.VMEM((16 * SD,), jnp.float32),
            pltpu.VMEM((16,), jnp.int32),
            pltpu.VMEM((32,), jnp.int32),
            pltpu.VMEM((128,), jnp.int32),
            pltpu.VMEM((128,), jnp.int32),
            pltpu.VMEM((128, MSGW), jnp.float32),
            pltpu.VMEM((128, MSGW), jnp.float32),
            pltpu.VMEM((256,), jnp.float32),
            pltpu.VMEM((16, 16), jnp.float32),
            pltpu.VMEM((16, 16), jnp.int32),
        ],
    )
    def knl(s_hbm, msg_hbm, batch_hbm, seg_hbm, agg_hbm,
            cand, qsb, qbb, segb, idxa, idxb, rowsa, rowsb, outb,
            bestd, besti):
        cid = lax.axis_index("c")
        sid = lax.axis_index("s")
        wid = sid * 2 + cid
        pltpu.sync_copy(seg_hbm, segb)
        q_start = wid * QPW
        ngroups = jnp.minimum(QPW, NPTS - q_start) // 16
        lane = lax.broadcasted_iota(jnp.int32, (16,), 0)

        def group_body(g, carry):
            q0 = q_start + g * 16
            pltpu.sync_copy(s_hbm.at[pl.ds(q0 * SD, 16 * SD)], qsb)
            pltpu.sync_copy(batch_hbm.at[pl.ds(q0, 16)], qbb)
            qb = qbb[...]
            lo = plsc.load_gather(segb, [qb])
            hi = plsc.load_gather(segb, [qb + 1])
            qs = [plsc.load_gather(qsb, [lane * SD + d]) for d in range(SD)]
            u_lo = jnp.min(lo)
            u_hi = jnp.max(hi)
            mx_lo = jnp.max(lo)
            mn_hi = jnp.min(hi)
            c_base = (u_lo // 8) * 8
            nch = (u_hi - c_base + (CH - 1)) // CH

            for k in range(16):
                bestd[k, :] = jnp.full((16,), INF, jnp.float32)
                besti[k, :] = jnp.zeros((16,), jnp.int32)

            def insert4(dists, c_first):
                obd = [bestd[k, :] for k in range(16)]
                obi = [besti[k, :] for k in range(16)]
                for t in range(4):
                    dist = dists[t]
                    cvec = jnp.full((16,), c_first + t, jnp.int32)
                    m = [dist < obd[k] for k in range(16)]
                    nd = [None] * 16
                    ni = [None] * 16
                    for k in range(15, -1, -1):
                        if k == 0:
                            sd_, si_ = dist, cvec
                        else:
                            sd_ = jnp.where(m[k - 1], obd[k - 1], dist)
                            si_ = jnp.where(m[k - 1], obi[k - 1], cvec)
                        nd[k] = jnp.where(m[k], sd_, obd[k])
                        ni[k] = jnp.where(m[k], si_, obi[k])
                    obd, obi = nd, ni
                for k in range(16):
                    bestd[k, :] = obd[k]
                    besti[k, :] = obi[k]

            def make_body(c0, checked):
                def body(jj, carry):
                    cv = cand[pl.ds(jj * 16, 16)]
                    worst = bestd[15, :]
                    dists = []
                    anym = None
                    for t in range(4):
                        acc = None
                        for d in range(SD):
                            b = _lane_bcast(
                                cv, jnp.full((16,), 4 * t + d, jnp.int32))
                            tt = qs[d] - b
                            acc = tt * tt if acc is None else acc + tt * tt
                        if checked:
                            cvec = jnp.full((16,), c0 + jj * 4 + t, jnp.int32)
                            valid = (cvec >= lo) & (cvec < hi)
                            acc = jnp.where(valid, acc, INF)
                        m = acc < worst
                        anym = m if anym is None else (anym | m)
                        dists.append(acc)
                    cnt = plsc.all_reduce_population_count(anym)

                    @pl.when(cnt[0] > 0)
                    def _():
                        insert4(dists, c0 + jj * 4)

                    return carry
                return body

            def chunk_body(ci, carry):
                c0 = c_base + ci * CH
                pltpu.sync_copy(s_hbm.at[pl.ds(c0 * SD, CH * SD)], cand)
                rel_lo = jnp.clip(u_lo - c0, 0, CH)
                rel_ml = jnp.clip(mx_lo - c0, 0, CH)
                rel_mh = jnp.clip(mn_hi - c0, 0, CH)
                rel_hi = jnp.clip(u_hi - c0, 0, CH)
                b0 = rel_lo // 4
                b3 = (rel_hi + 3) // 4
                b1 = jnp.minimum((rel_ml + 3) // 4, b3)
                b1 = jnp.maximum(b0, b1)
                b2 = jnp.maximum(jnp.minimum(rel_mh // 4, b3), b1)
                lax.fori_loop(b0, b1, make_body(c0, True), 0)
                lax.fori_loop(b1, b2, make_body(c0, False), 0)
                lax.fori_loop(b2, b3, make_body(c0, True), 0)
                return carry

            lax.fori_loop(0, nch, chunk_body, 0)
            bd = [bestd[k, :] for k in range(16)]
            bi = [besti[k, :] for k in range(16)]
            ews = [jnp.exp(-10.0 * bd[k]) for k in range(16)]
            for k in range(16):
                if k < 8:
                    idxa[pl.ds(k * 16, 16)] = bi[k]
                else:
                    idxb[pl.ds((k - 8) * 16, 16)] = bi[k]
            pltpu.sync_copy(msg_hbm.at[idxa], rowsa)
            pltpu.sync_copy(msg_hbm.at[idxb], rowsb)
            for ql in range(16):
                a = jnp.zeros((16,), jnp.float32)
                qlv = jnp.full((16,), ql, jnp.int32)
                for k in range(16):
                    rref = rowsa if k < 8 else rowsb
                    w = _lane_bcast(ews[k], qlv)
                    a = a + w * rref[(k % 8) * 16 + ql, pl.ds(0, 16)]
                outb[pl.ds(ql * 16, 16)] = a
            pltpu.sync_copy(outb, agg_hbm.at[pl.ds(q0 * PDP, 256)])
            return carry

        lax.fori_loop(0, ngroups, group_body, 0)

    return knl(s_flat, msg16, batch, seg)


def _prep_lin(p):
    return jnp.asarray(p["W"]).T, jnp.asarray(p["b"])[None, :]


def _prep_lin_padout(p, out_pad):
    w, b = _prep_lin(p)
    w = jnp.pad(w, ((0, 0), (0, out_pad - w.shape[1])))
    b = jnp.pad(b, ((0, 0), (0, out_pad - b.shape[1])))
    return w, b


def kernel(x, batch, params):
    batch = batch.astype(jnp.int32)
    seg = jnp.searchsorted(
        batch, jnp.arange(NBATCH + 1, dtype=jnp.int32), side="left"
    ).astype(jnp.int32)
    seg = jnp.pad(seg, (0, 32 - (NBATCH + 1)))

    nn1 = params["nn1"]
    convs = params["convs"]
    nn2 = params["nn2"]
    nn3 = params["nn3"]

    h = x
    for lp in nn1[:-1]:
        w, b = _prep_lin(lp)
        h = jax.nn.elu(_tc_lin(h, w, b))
    w4, b4 = _prep_lin(nn1[-1])
    wp, bp = _prep_lin_padout(convs[0]["lin_p"], MSGW)
    ws, bs = _prep_lin(convs[0]["lin_s"])
    msg16, s = _tc_emb3(h, [w4, b4, wp, bp, ws, bs])

    for i in range(3):
        s_flat = jnp.pad(s, ((0, CH), (0, 0))).reshape(-1)
        agg_flat = _sc_gravnet(s_flat, msg16, batch, seg)
        agg16 = agg_flat.reshape(NPTS, PDP)
        wo, bo = _prep_lin(convs[i]["lin_out"])
        wo = jnp.pad(wo, ((0, PDP - PD), (0, 0)))
        if i < 2:
            wp, bp = _prep_lin_padout(convs[i + 1]["lin_p"], MSGW)
            ws, bs = _prep_lin(convs[i + 1]["lin_s"])
            msg16, s = _tc_mid(agg16, [wo, bo, wp, bp, ws, bs])
        else:
            w1, b1 = _prep_lin(nn2[0])
            p_post = [wo, bo,
                      w1[:12], w1[12:], b1]
            for lp in nn2[1:]:
                w, b = _prep_lin(lp)
                p_post += [w, b]
            v1, c1 = _prep_lin(nn3[0])
            p_post += [v1[:12], v1[12:], c1]
            for lp in nn3[1:]:
                w, b = _prep_lin(lp)
                p_post += [w, b]
            preds_id, preds_p4 = _tc_post(agg16, x, p_post)
    return preds_id, preds_p4


# speedup vs baseline: 1.3977x; 1.0039x over previous
import functools

import jax
import jax.numpy as jnp
from jax import lax
from jax.experimental import pallas as pl
from jax.experimental.pallas import tpu as pltpu
from jax.experimental.pallas import tpu_sc as plsc

NPTS = 50000
NBATCH = 16
KNN = 16
SD = 4
PD = 8
PDP = 16
MSGW = 128
EMBD = 32
CH = 4096
NWORK = 32
QPW = 1568
ROWS = 2000


def _elu(h):
    small = jnp.abs(h) < 1e-5
    em1 = jnp.where(small, h + 0.5 * h * h, jnp.exp(h) - 1.0)
    return jnp.where(h > 0.0, h, em1)


def _mm(a, b):
    return jnp.dot(a, b)


def _full_spec(shape):
    return pl.BlockSpec(shape, lambda i: tuple(0 for _ in shape))


def _row_spec(cols):
    return pl.BlockSpec((ROWS, cols), lambda i: (i, 0))


def _tc_lin(h, w, b):
    def body(h_ref, w_ref, b_ref, o_ref):
        o_ref[...] = _mm(h_ref[...], w_ref[...]) + b_ref[...]

    return pl.pallas_call(
        body,
        grid=(NPTS // ROWS,),
        in_specs=[_row_spec(h.shape[1]), _full_spec(w.shape),
                  _full_spec(b.shape)],
        out_specs=_row_spec(w.shape[1]),
        out_shape=jax.ShapeDtypeStruct((NPTS, w.shape[1]), jnp.float32),
    )(h, w, b)


def _tc_emb3(h, p):
    def body(h_ref, w4, b4, wp, bp, ws, bs, msg_ref, s_ref):
        e = _mm(h_ref[...], w4[...]) + b4[...]
        msg_ref[...] = _mm(e, wp[...]) + bp[...]
        s_ref[...] = _mm(e, ws[...]) + bs[...]

    args = [h] + p
    return pl.pallas_call(
        body,
        grid=(NPTS // ROWS,),
        in_specs=[_row_spec(h.shape[1])] + [_full_spec(a.shape) for a in p],
        out_specs=[_row_spec(MSGW), _row_spec(SD)],
        out_shape=[jax.ShapeDtypeStruct((NPTS, MSGW), jnp.float32),
                   jax.ShapeDtypeStruct((NPTS, SD), jnp.float32)],
    )(*args)


def _tc_mid(agg16, p):
    def body(a_ref, wo, bo, wp, bp, ws, bs, msg_ref, s_ref):
        e = _mm(a_ref[...], wo[...]) + bo[...]
        msg_ref[...] = _mm(e, wp[...]) + bp[...]
        s_ref[...] = _mm(e, ws[...]) + bs[...]

    args = [agg16] + p
    return pl.pallas_call(
        body,
        grid=(NPTS // ROWS,),
        in_specs=[_row_spec(PDP)] + [_full_spec(a.shape) for a in p],
        out_specs=[_row_spec(MSGW), _row_spec(SD)],
        out_shape=[jax.ShapeDtypeStruct((NPTS, MSGW), jnp.float32),
                   jax.ShapeDtypeStruct((NPTS, SD), jnp.float32)],
    )(*args)


def _tc_post(agg16, x, p):
    def body(a_ref, x_ref, wo, bo,
             w1x, w1e, b1, w2, b2, w3, b3, w4, b4,
             v1x, v1p, c1, v2, c2, v3, c3, v4, c4,
             id_ref, p4_ref):
        xb = x_ref[...]
        e = _mm(a_ref[...], wo[...]) + bo[...]
        h = _elu(_mm(xb, w1x[...]) + _mm(e, w1e[...]) + b1[...])
        h = _elu(_mm(h, w2[...]) + b2[...])
        h = _elu(_mm(h, w3[...]) + b3[...])
        pid = _mm(h, w4[...]) + b4[...]
        id_ref[...] = pid
        h = _elu(_mm(xb, v1x[...]) + _mm(pid, v1p[...]) + c1[...])
        h = _elu(_mm(h, v2[...]) + c2[...])
        h = _elu(_mm(h, v3[...]) + c3[...])
        p4_ref[...] = _mm(h, v4[...]) + c4[...]

    args = [agg16, x] + p
    return pl.pallas_call(
        body,
        grid=(NPTS // ROWS,),
        in_specs=[_row_spec(PDP), _row_spec(x.shape[1])]
        + [_full_spec(a.shape) for a in p],
        out_specs=[_row_spec(6), _row_spec(6)],
        out_shape=[jax.ShapeDtypeStruct((NPTS, 6), jnp.float32),
                   jax.ShapeDtypeStruct((NPTS, 6), jnp.float32)],
    )(*args)


def _lane_bcast(v, idx):
    dn = lax.GatherDimensionNumbers(
        offset_dims=(), collapsed_slice_dims=(0,), start_index_map=(0,))
    return lax.gather(v, idx[:, None], dn, slice_sizes=(1,),
                      mode=lax.GatherScatterMode.PROMISE_IN_BOUNDS)


def _sc_compiler_params():
    import dataclasses
    cp = pltpu.CompilerParams()
    if "needs_layout_passes" in pltpu.CompilerParams.__dataclass_fields__:
        cp = dataclasses.replace(cp, needs_layout_passes=False)
    return cp


def _sc_gravnet(s_flat, msg16, batch, seg):
    mesh = plsc.VectorSubcoreMesh(core_axis_name="c", subcore_axis_name="s")
    INF = jnp.float32(1e30)

    @functools.partial(
        pl.kernel,
        out_type=jax.ShapeDtypeStruct((NPTS * PDP,), jnp.float32),
        mesh=mesh,
        compiler_params=_sc_compiler_params(),
        scratch_types=[
            pltpu.VMEM((CH * SD,), jnp.float32),
            pltpu.VMEM((16 * SD,), jnp.float32),
            pltpu.VMEM((16,), jnp.int32),
            pltpu.VMEM((32,), jnp.int32),
            pltpu.VMEM((128,), jnp.int32),
            pltpu.VMEM((128,), jnp.int32),
            pltpu.VMEM((128, MSGW), jnp.float32),
            pltpu.VMEM((128, MSGW), jnp.float32),
            pltpu.VMEM((256,), jnp.float32),
            pltpu.VMEM((16, 16), jnp.float32),
            pltpu.VMEM((16, 16), jnp.int32),
        ],
    )
    def knl(s_hbm, msg_hbm, batch_hbm, seg_hbm, agg_hbm,
            cand, qsb, qbb, segb, idxa, idxb, rowsa, rowsb, outb,
            bestd, besti):
        cid = lax.axis_index("c")
        sid = lax.axis_index("s")
        wid = sid * 2 + cid
        pltpu.sync_copy(seg_hbm, segb)
        q_start = wid * QPW
        ngroups = jnp.minimum(QPW, NPTS - q_start) // 16
        lane = lax.broadcasted_iota(jnp.int32, (16,), 0)

        def group_body(g, carry):
            q0 = q_start + g * 16
            pltpu.sync_copy(s_hbm.at[pl.ds(q0 * SD, 16 * SD)], qsb)
            pltpu.sync_copy(batch_hbm.at[pl.ds(q0, 16)], qbb)
            qb = qbb[...]
            lo = plsc.load_gather(segb, [qb])
            hi = plsc.load_gather(segb, [qb + 1])
            qs = [plsc.load_gather(qsb, [lane * SD + d]) for d in range(SD)]
            u_lo = jnp.min(lo)
            u_hi = jnp.max(hi)
            mx_lo = jnp.max(lo)
            mn_hi = jnp.min(hi)
            c_base = (u_lo // 8) * 8
            nch = (u_hi - c_base + (CH - 1)) // CH

            for k in range(16):
                bestd[k, :] = jnp.full((16,), INF, jnp.float32)
                besti[k, :] = jnp.zeros((16,), jnp.int32)

            def insert4(dists, c_first):
                obd = [bestd[k, :] for k in range(16)]
                obi = [besti[k, :] for k in range(16)]
                for t in range(4):
                    dist = dists[t]
                    cvec = jnp.full((16,), c_first + t, jnp.int32)
                    m = [dist < obd[k] for k in range(16)]
                    nd = [None] * 16
                    ni = [None] * 16
                    for k in range(15, -1, -1):
                        if k == 0:
                            sd_, si_ = dist, cvec
                        else:
                            sd_ = jnp.where(m[k - 1], obd[k - 1], dist)
                            si_ = jnp.where(m[k - 1], obi[k - 1], cvec)
                        nd[k] = jnp.where(m[k], sd_, obd[k])
                        ni[k] = jnp.where(m[k], si_, obi[k])
                    obd, obi = nd, ni
                for k in range(16):
                    bestd[k, :] = obd[k]
                    besti[k, :] = obi[k]

            def make_body(c0, checked):
                def body(jj, carry):
                    worst = bestd[15, :]
                    dists = []
                    anym = None
                    subany = []
                    for u in range(4):
                        cv = cand[pl.ds(jj * 64 + u * 16, 16)]
                        um = None
                        for t in range(4):
                            acc = None
                            for d in range(SD):
                                b = _lane_bcast(
                                    cv, jnp.full((16,), 4 * t + d, jnp.int32))
                                tt = qs[d] - b
                                acc = tt * tt if acc is None else acc + tt * tt
                            if checked:
                                cvec = jnp.full(
                                    (16,), c0 + jj * 16 + u * 4 + t, jnp.int32)
                                valid = (cvec >= lo) & (cvec < hi)
                                acc = jnp.where(valid, acc, INF)
                            m = acc < worst
                            um = m if um is None else (um | m)
                            dists.append(acc)
                        subany.append(um)
                        anym = um if anym is None else (anym | um)
                    cnt = plsc.all_reduce_population_count(anym)

                    @pl.when(cnt[0] > 0)
                    def _():
                        for u in range(4):
                            w2 = bestd[15, :]
                            m2 = ((dists[4 * u] < w2) | (dists[4 * u + 1] < w2)
                                  | (dists[4 * u + 2] < w2)
                                  | (dists[4 * u + 3] < w2))
                            c2 = plsc.all_reduce_population_count(m2)

                            @pl.when(c2[0] > 0)
                            def _(u=u):
                                insert4(dists[4 * u:4 * u + 4],
                                        c0 + jj * 16 + u * 4)

                    return carry
                return body

            def chunk_body(ci, carry):
                c0 = c_base + ci * CH
                pltpu.sync_copy(s_hbm.at[pl.ds(c0 * SD, CH * SD)], cand)
                rel_lo = jnp.clip(u_lo - c0, 0, CH)
                rel_ml = jnp.clip(mx_lo - c0, 0, CH)
                rel_mh = jnp.clip(mn_hi - c0, 0, CH)
                rel_hi = jnp.clip(u_hi - c0, 0, CH)
                b0 = rel_lo // 16
                b3 = (rel_hi + 15) // 16
                b1 = jnp.minimum((rel_ml + 15) // 16, b3)
                b1 = jnp.maximum(b0, b1)
                b2 = jnp.maximum(jnp.minimum(rel_mh // 16, b3), b1)
                lax.fori_loop(b0, b1, make_body(c0, True), 0)
                lax.fori_loop(b1, b2, make_body(c0, False), 0)
                lax.fori_loop(b2, b3, make_body(c0, True), 0)
                return carry

            lax.fori_loop(0, nch, chunk_body, 0)
            bd = [bestd[k, :] for k in range(16)]
            bi = [besti[k, :] for k in range(16)]
            ews = [jnp.exp(-10.0 * bd[k]) for k in range(16)]
            for k in range(16):
                if k < 8:
                    idxa[pl.ds(k * 16, 16)] = bi[k]
                else:
                    idxb[pl.ds((k - 8) * 16, 16)] = bi[k]
            pltpu.sync_copy(msg_hbm.at[idxa], rowsa)
            pltpu.sync_copy(msg_hbm.at[idxb], rowsb)
            for ql in range(16):
                a = jnp.zeros((16,), jnp.float32)
                qlv = jnp.full((16,), ql, jnp.int32)
                for k in range(16):
                    rref = rowsa if k < 8 else rowsb
                    w = _lane_bcast(ews[k], qlv)
                    a = a + w * rref[(k % 8) * 16 + ql, pl.ds(0, 16)]
                outb[pl.ds(ql * 16, 16)] = a
            pltpu.sync_copy(outb, agg_hbm.at[pl.ds(q0 * PDP, 256)])
            return carry

        lax.fori_loop(0, ngroups, group_body, 0)

    return knl(s_flat, msg16, batch, seg)


def _prep_lin(p):
    return jnp.asarray(p["W"]).T, jnp.asarray(p["b"])[None, :]


def _prep_lin_padout(p, out_pad):
    w, b = _prep_lin(p)
    w = jnp.pad(w, ((0, 0), (0, out_pad - w.shape[1])))
    b = jnp.pad(b, ((0, 0), (0, out_pad - b.shape[1])))
    return w, b


def kernel(x, batch, params):
    batch = batch.astype(jnp.int32)
    seg = jnp.searchsorted(
        batch, jnp.arange(NBATCH + 1, dtype=jnp.int32), side="left"
    ).astype(jnp.int32)
    seg = jnp.pad(seg, (0, 32 - (NBATCH + 1)))

    nn1 = params["nn1"]
    convs = params["convs"]
    nn2 = params["nn2"]
    nn3 = params["nn3"]

    h = x
    for lp in nn1[:-1]:
        w, b = _prep_lin(lp)
        h = jax.nn.elu(_tc_lin(h, w, b))
    w4, b4 = _prep_lin(nn1[-1])
    wp, bp = _prep_lin_padout(convs[0]["lin_p"], MSGW)
    ws, bs = _prep_lin(convs[0]["lin_s"])
    msg16, s = _tc_emb3(h, [w4, b4, wp, bp, ws, bs])

    for i in range(3):
        s_flat = jnp.pad(s, ((0, CH), (0, 0))).reshape(-1)
        agg_flat = _sc_gravnet(s_flat, msg16, batch, seg)
        agg16 = agg_flat.reshape(NPTS, PDP)
        wo, bo = _prep_lin(convs[i]["lin_out"])
        wo = jnp.pad(wo, ((0, PDP - PD), (0, 0)))
        if i < 2:
            wp, bp = _prep_lin_padout(convs[i + 1]["lin_p"], MSGW)
            ws, bs = _prep_lin(convs[i + 1]["lin_s"])
            msg16, s = _tc_mid(agg16, [wo, bo, wp, bp, ws, bs])
        else:
            w1, b1 = _prep_lin(nn2[0])
            p_post = [wo, bo,
                      w1[:12], w1[12:], b1]
            for lp in nn2[1:]:
                w, b = _prep_lin(lp)
                p_post += [w, b]
            v1, c1 = _prep_lin(nn3[0])
            p_post += [v1[:12], v1[12:], c1]
            for lp in nn3[1:]:
                w, b = _prep_lin(lp)
                p_post += [w, b]
            preds_id, preds_p4 = _tc_post(agg16, x, p_post)
    return preds_id, preds_p4
